# baseline (speedup 1.0000x reference)
import functools

import jax
import jax.numpy as jnp
from jax import lax
from jax.experimental import pallas as pl
from jax.experimental.pallas import tpu as pltpu
from jax.experimental.pallas import tpu_sc as plsc

N = 10000
E = 160000
DM = 128
PD = 4
NHEADS = 8
HD = 16
NGRAPH = 64

NPAD = 10240
EPAD = 163840
NW = 32
CHK = 128
CPW = EPAD // (NW * CHK)
RB = 512
NB = NPAD // RB
EB = 512
STRIPE = NPAD // 16

_f32 = jnp.float32


def _sigmoid(x):
    return 1.0 / (1.0 + jnp.exp(-x))


def _gelu_tanh(x):
    return 0.5 * x * (1.0 + jnp.tanh(0.7978845608028654 * (x + 0.044715 * x * x * x)))


TW = 2 * DM


def _build_sc_kernels():
    mesh = plsc.VectorSubcoreMesh(core_axis_name="c", subcore_axis_name="s")

    GCHK = 64
    GPW = EPAD // (NW * GCHK)
    EPW = GPW * GCHK

    @functools.partial(
        pl.kernel,
        mesh=mesh,
        out_type=[
            jax.ShapeDtypeStruct((EPAD, TW), _f32),
            jax.ShapeDtypeStruct((EPAD, TW), _f32),
        ],
        scratch_types=[
            pltpu.VMEM((EPW,), jnp.int32),
            pltpu.VMEM((EPW,), jnp.int32),
            pltpu.VMEM((2, GCHK, TW), _f32),
            pltpu.VMEM((2, GCHK, TW), _f32),
            pltpu.SemaphoreType.DMA,
            pltpu.SemaphoreType.DMA,
            pltpu.SemaphoreType.DMA,
            pltpu.SemaphoreType.DMA,
        ],
    )
    def gather(tbl, rowi, coli, gr, gc, idr, idc, bufr, bufc, g0, g1, w0, w1):
        wid = lax.axis_index("s") * 2 + lax.axis_index("c")
        base = wid * EPW
        gsem = (g0, g1)
        wsem = (w0, w1)
        pltpu.sync_copy(rowi.at[pl.ds(base, EPW)], idr)
        pltpu.sync_copy(coli.at[pl.ds(base, EPW)], idc)

        def issue(j, slot):
            pltpu.async_copy(tbl.at[idr.at[pl.ds(j * GCHK, GCHK)]],
                             bufr.at[slot], gsem[slot])
            pltpu.async_copy(tbl.at[idc.at[pl.ds(j * GCHK, GCHK)]],
                             bufc.at[slot], gsem[slot])

        def drain(j, slot):
            pltpu.make_async_copy(tbl.at[idr.at[pl.ds(j * GCHK, GCHK)]],
                                  bufr.at[slot], gsem[slot]).wait()
            pltpu.make_async_copy(tbl.at[idc.at[pl.ds(j * GCHK, GCHK)]],
                                  bufc.at[slot], gsem[slot]).wait()

        def issue_wb(j, slot):
            off = base + j * GCHK
            pltpu.async_copy(bufr.at[slot], gr.at[pl.ds(off, GCHK)], wsem[slot])
            pltpu.async_copy(bufc.at[slot], gc.at[pl.ds(off, GCHK)], wsem[slot])

        def drain_wb(j, slot):
            off = base + j * GCHK
            pltpu.make_async_copy(bufr.at[slot], gr.at[pl.ds(off, GCHK)],
                                  wsem[slot]).wait()
            pltpu.make_async_copy(bufc.at[slot], gc.at[pl.ds(off, GCHK)],
                                  wsem[slot]).wait()

        issue(0, 0)

        def step(ci, slot):
            j = ci * 2 + slot

            @pl.when(j + 1 < GPW)
            def _():
                @pl.when(j >= 1)
                def _():
                    drain_wb(j - 1, 1 - slot)

                issue(j + 1, 1 - slot)

            drain(j, slot)
            issue_wb(j, slot)

        def body(ci, carry):
            step(ci, 0)
            step(ci, 1)
            return carry

        lax.fori_loop(0, GPW // 2, body, 0)
        drain_wb(GPW - 2, 0)
        drain_wb(GPW - 1, 1)

    HALF = NPAD // 2
    ACCR = 5248
    ZST = ACCR // 16
    OST = HALF // 16
    CPT = EPAD // (16 * CHK)

    @functools.partial(
        pl.kernel,
        mesh=mesh,
        out_type=[jax.ShapeDtypeStruct((NPAD, DM), _f32)],
        scratch_types=[
            pltpu.VMEM((CPT, CHK), jnp.int32),
            pltpu.VMEM((2, CHK, DM), _f32),
            pltpu.VMEM((ZST, DM), _f32),
            pltpu.VMEM_SHARED((ACCR, DM), _f32),
            pltpu.SemaphoreType.DMA,
            pltpu.SemaphoreType.DMA,
            pltpu.SemaphoreType.DMA,
            pltpu.SemaphoreType.DMA,
        ],
    )
    def scatter(msgs, coli, out, idx, sbuf, zbuf, acc, sem0, sem1, a0, a1):
        cid = lax.axis_index("c")
        sid = lax.axis_index("s")
        lo = cid * HALF
        sems = (sem0, sem1)
        asem = (a0, a1)
        pltpu.sync_copy(msgs.at[pl.ds(E, ZST)], zbuf)
        pltpu.sync_copy(zbuf, acc.at[pl.ds(sid * ZST, ZST)])
        pltpu.sync_copy(coli.at[pl.ds(sid * CPT, CPT)], idx)

        def remap(j, carry):
            r = idx.at[j]
            for g in range(CHK // 16):
                v = r[pl.ds(g * 16, 16)] - lo
                inr = (v >= 0) & (v < HALF)
                r[pl.ds(g * 16, 16)] = jnp.where(inr, v, HALF)
            return carry

        lax.fori_loop(0, CPT, remap, 0)
        plsc.subcore_barrier()

        def issue(j, slot):
            pltpu.async_copy(msgs.at[pl.ds((sid * CPT + j) * CHK, CHK)],
                             sbuf.at[slot], sems[slot])

        def drain(j, slot):
            pltpu.make_async_copy(msgs.at[pl.ds((sid * CPT + j) * CHK, CHK)],
                                  sbuf.at[slot], sems[slot]).wait()

        def issue_add(j, slot):
            pltpu.async_copy(sbuf.at[slot], acc.at[idx.at[j]], asem[slot],
                             add=True)

        def drain_add(j, slot):
            pltpu.make_async_copy(sbuf.at[slot], acc.at[idx.at[j]],
                                  asem[slot]).wait()

        issue(0, 0)

        def step(ci, slot):
            j = ci * 2 + slot

            @pl.when(j + 1 < CPT)
            def _():
                @pl.when(j >= 1)
                def _():
                    drain_add(j - 1, 1 - slot)

                issue(j + 1, 1 - slot)

            drain(j, slot)
            issue_add(j, slot)

        def body(ci, carry):
            step(ci, 0)
            step(ci, 1)
            return carry

        lax.fori_loop(0, CPT // 2, body, 0)
        drain_add(CPT - 2, 0)
        drain_add(CPT - 1, 1)
        plsc.subcore_barrier()
        pltpu.sync_copy(acc.at[pl.ds(sid * OST, OST)],
                        out.at[pl.ds(lo + sid * OST, OST)])

    return gather, scatter


_SC_CACHE = {}


def _sc_gather(tbl, rowp, colp):
    if "k" not in _SC_CACHE:
        _SC_CACHE["k"] = _build_sc_kernels()
    return _SC_CACHE["k"][0](tbl, rowp, colp)


def _sc_scatter(msgs, colp2):
    if "k" not in _SC_CACHE:
        _SC_CACHE["k"] = _build_sc_kernels()
    out = _SC_CACHE["k"][1](msgs, colp2)
    if isinstance(out, (tuple, list)):
        out = out[0]
    return out


def _edge_body(gr_ref, gc_ref, w1a, w1b, w1c, b1, w2, b2, pw1, pb1,
               pw2, pb2, m_ref, pm_ref):
    gr = gr_ref[...]
    gc = gc_ref[...]
    hr = gr[:, :DM]
    hc = gc[:, :DM]
    rel = gr[:, DM:DM + PD] - gc[:, DM:DM + PD]
    dist = jnp.sqrt(jnp.sum(rel * rel, axis=1, keepdims=True))
    z = (jnp.dot(hr, w1a[...], preferred_element_type=_f32)
         + jnp.dot(hc, w1b[...], preferred_element_type=_f32)
         + dist * w1c[...] + b1[...])
    t = z * _sigmoid(z)
    emsg = jnp.dot(t, w2[...], preferred_element_type=_f32) + b2[...]
    u = emsg @ pw1[...] + pb1[...]
    u = u * _sigmoid(u)
    coeff = jnp.dot(u, pw2[...], preferred_element_type=_f32) + pb2[...]
    gid = pl.program_id(0) * EB + lax.broadcasted_iota(jnp.int32, (EB, 1), 0)
    valid = gid < E
    m_ref[...] = jnp.where(valid, hr * emsg, 0.0)
    pm = jnp.where(valid, rel * coeff, 0.0)
    pm_ref[...] = jnp.concatenate([pm, jnp.zeros((EB, DM - PD), _f32)], axis=1)


def _edge_mlp(gr, gc, wts):
    full = lambda shape: pl.BlockSpec(shape, lambda i: (0,) * len(shape))
    return pl.pallas_call(
        _edge_body,
        grid=(EPAD // EB,),
        in_specs=[
            pl.BlockSpec((EB, TW), lambda i: (i, 0)),
            pl.BlockSpec((EB, TW), lambda i: (i, 0)),
            full((DM, DM)), full((DM, DM)), full((1, DM)), full((1, DM)),
            full((DM, DM)), full((1, DM)),
            full((DM, DM)), full((1, DM)), full((DM, 1)), full((1, 1)),
        ],
        out_specs=[pl.BlockSpec((EB, DM), lambda i: (i, 0)),
                   pl.BlockSpec((EB, DM), lambda i: (i, 0))],
        out_shape=[jax.ShapeDtypeStruct((EPAD, DM), _f32),
                   jax.ShapeDtypeStruct((EPAD, DM), _f32)],
    )(gr, gc, *wts)


def _ln1_body(h_ref, sp_ref, pp_ref, posp_ref, wq, wk, wv, bq, bk, bv, g1, b1g,
              hn_ref, q_ref, k_ref, v_ref, pos_ref):
    msg = sp_ref[...]
    pm = pp_ref[:, :PD]
    t = h_ref[...] + msg
    mu = jnp.mean(t, axis=1, keepdims=True)
    d = t - mu
    var = jnp.mean(d * d, axis=1, keepdims=True)
    hn = d * lax.rsqrt(var + 1e-5) * g1[...] + b1g[...]
    hn_ref[...] = hn
    q_ref[...] = jnp.dot(hn, wq[...], preferred_element_type=_f32) + bq[...]
    k_ref[...] = jnp.dot(hn, wk[...], preferred_element_type=_f32) + bk[...]
    v_ref[...] = jnp.dot(hn, wv[...], preferred_element_type=_f32) + bv[...]
    pos_ref[...] = posp_ref[...] + pm


def _ln1_qkv(h, sp, pp, posp, wts):
    full = lambda shape: pl.BlockSpec(shape, lambda i: (0,) * len(shape))
    blk = pl.BlockSpec((RB, DM), lambda i: (i, 0))
    return pl.pallas_call(
        _ln1_body,
        grid=(NB,),
        in_specs=[
            blk,
            blk,
            blk,
            pl.BlockSpec((RB, PD), lambda i: (i, 0)),
            full((DM, DM)), full((DM, DM)), full((DM, DM)),
            full((1, DM)), full((1, DM)), full((1, DM)),
            full((1, DM)), full((1, DM)),
        ],
        out_specs=[blk, blk, blk, blk, pl.BlockSpec((RB, PD), lambda i: (i, 0))],
        out_shape=[
            jax.ShapeDtypeStruct((NPAD, DM), _f32),
            jax.ShapeDtypeStruct((NPAD, DM), _f32),
            jax.ShapeDtypeStruct((NPAD, DM), _f32),
            jax.ShapeDtypeStruct((NPAD, DM), _f32),
            jax.ShapeDtypeStruct((NPAD, PD), _f32),
        ],
    )(h, sp, pp, posp, *wts)


def _attn_body(jb_ref, q_ref, k3_ref, v3_ref, br_ref, bc3_ref, o_ref):
    i = pl.program_id(0)
    jlo = jb_ref[i, 0]
    jhi = jb_ref[i, 1]
    q = q_ref[...]
    br = br_ref[...]

    carry = []
    for _ in range(NHEADS):
        carry.append(jnp.full((RB, 1), -1e30, _f32))
        carry.append(jnp.zeros((RB, 1), _f32))
        carry.append(jnp.zeros((RB, HD), _f32))
    carry = tuple(carry)

    def body(j, carry):
        kc = k3_ref[j]
        vc = v3_ref[j]
        bc = bc3_ref[j]
        mask = br == bc
        out = []
        for h in range(NHEADS):
            m_h = carry[3 * h]
            l_h = carry[3 * h + 1]
            a_h = carry[3 * h + 2]
            qh = q[:, h * HD:(h + 1) * HD]
            kh = kc[:, h * HD:(h + 1) * HD]
            vh = vc[:, h * HD:(h + 1) * HD]
            s = lax.dot_general(qh, kh, (((1,), (1,)), ((), ())),
                                preferred_element_type=_f32) * 0.25
            s = jnp.where(mask, s, -1e9)
            mnew = jnp.maximum(m_h, jnp.max(s, axis=1, keepdims=True))
            alpha = jnp.exp(m_h - mnew)
            p = jnp.exp(s - mnew)
            l_new = l_h * alpha + jnp.sum(p, axis=1, keepdims=True)
            a_new = a_h * alpha + jnp.dot(p, vh, preferred_element_type=_f32)
            out.extend([mnew, l_new, a_new])
        return tuple(out)

    carry = lax.fori_loop(jlo, jhi, body, carry)
    outs = [carry[3 * h + 2] / carry[3 * h + 1] for h in range(NHEADS)]
    o_ref[...] = jnp.concatenate(outs, axis=1)


def _attention(q, k3, v3, br, bc3, jb):
    spec = pltpu.PrefetchScalarGridSpec(
        num_scalar_prefetch=1,
        grid=(NB,),
        in_specs=[
            pl.BlockSpec((RB, DM), lambda i, jb: (i, 0)),
            pl.BlockSpec((NB, RB, DM), lambda i, jb: (0, 0, 0)),
            pl.BlockSpec((NB, RB, DM), lambda i, jb: (0, 0, 0)),
            pl.BlockSpec((RB, 1), lambda i, jb: (i, 0)),
            pl.BlockSpec((NB, 1, RB), lambda i, jb: (0, 0, 0)),
        ],
        out_specs=pl.BlockSpec((RB, DM), lambda i, jb: (i, 0)),
    )
    return pl.pallas_call(
        _attn_body,
        grid_spec=spec,
        out_shape=jax.ShapeDtypeStruct((NPAD, DM), _f32),
    )(jb, q, k3, v3, br, bc3)


def _ff_body(hn_ref, a_ref, wo, bo, g2, b2g, w1, bb1, w2, bb2, out_ref):
    r = hn_ref[...] + jnp.dot(a_ref[...], wo[...], preferred_element_type=_f32) + bo[...]
    mu = jnp.mean(r, axis=1, keepdims=True)
    d = r - mu
    var = jnp.mean(d * d, axis=1, keepdims=True)
    h2 = d * lax.rsqrt(var + 1e-5) * g2[...] + b2g[...]
    f = _gelu_tanh(jnp.dot(h2, w1[...], preferred_element_type=_f32) + bb1[...])
    out_ref[...] = h2 + jnp.dot(f, w2[...], preferred_element_type=_f32) + bb2[...]


def _ff(hn, attn, wts):
    full = lambda shape: pl.BlockSpec(shape, lambda i: (0,) * len(shape))
    blk = pl.BlockSpec((RB, DM), lambda i: (i, 0))
    return pl.pallas_call(
        _ff_body,
        grid=(NB,),
        in_specs=[
            blk, blk,
            full((DM, DM)), full((1, DM)), full((1, DM)), full((1, DM)),
            full((DM, 4 * DM)), full((1, 4 * DM)),
            full((4 * DM, DM)), full((1, DM)),
        ],
        out_specs=blk,
        out_shape=jax.ShapeDtypeStruct((NPAD, DM), _f32),
    )(hn, attn, *wts)


def kernel(x, pos, edge_index, edge_attr, batch, params):
    del edge_attr
    row = edge_index[0].astype(jnp.int32)
    col = edge_index[1].astype(jnp.int32)
    epad = EPAD - E
    rowp = jnp.concatenate([row, jnp.zeros((epad,), jnp.int32)])
    colp = jnp.concatenate([col, jnp.zeros((epad,), jnp.int32)])
    colp2 = colp.reshape(EPAD // CHK, CHK)

    npd = NPAD - N
    h = jnp.concatenate([x, jnp.zeros((npd, DM), _f32)], axis=0)
    posp = jnp.concatenate(
        [jnp.pad(pos, ((0, 0), (0, PD - 3))), jnp.zeros((npd, PD), _f32)], axis=0)
    batch_pad = jnp.concatenate([batch.astype(jnp.int32),
                                 jnp.full((npd,), 127, jnp.int32)])

    starts = jnp.searchsorted(batch, jnp.arange(NGRAPH + 1)).astype(jnp.int32)
    cs = jnp.concatenate([starts[jnp.clip(batch, 0, NGRAPH - 1)],
                          jnp.full((npd,), N, jnp.int32)])
    ce = jnp.concatenate([starts[jnp.clip(batch, 0, NGRAPH - 1) + 1],
                          jnp.full((npd,), NPAD, jnp.int32)])
    r0 = jnp.arange(NB, dtype=jnp.int32) * RB
    jlo = cs[r0] // RB
    jhi = (ce[r0 + RB - 1] + RB - 1) // RB
    jb = jnp.stack([jlo, jhi], axis=1).astype(jnp.int32)

    br = batch_pad[:, None]
    bc3 = batch_pad.reshape(NB, 1, RB)
    ztail = jnp.zeros((NPAD, TW - DM - PD), _f32)

    for p in params:
        w1 = p['pe_W1']
        edge_wts = (
            w1[:, :DM].T, w1[:, DM:2 * DM].T, w1[:, 2 * DM:].T,
            p['pe_b1'][None], p['pe_W2'].T, p['pe_b2'][None],
            p['px_W1'].T, p['px_b1'][None], p['px_W2'].T, p['px_b2'][None],
        )
        wqkv = p['Wqkv']
        ln1_wts = (
            wqkv[:DM].T, wqkv[DM:2 * DM].T, wqkv[2 * DM:].T,
            p['bqkv'][None, :DM], p['bqkv'][None, DM:2 * DM], p['bqkv'][None, 2 * DM:],
            p['ln1_g'][None], p['ln1_b'][None],
        )
        ff_wts = (
            p['Wo'].T, p['bo'][None], p['ln2_g'][None], p['ln2_b'][None],
            p['ff_W1'].T, p['ff_b1'][None], p['ff_W2'].T, p['ff_b2'][None],
        )

        tbl = jnp.concatenate([h, posp, ztail], axis=1)
        gr, gc = _sc_gather(tbl, rowp, colp)
        msgs, pmsgs = _edge_mlp(gr, gc, edge_wts)
        hm = _sc_scatter(msgs, colp2)
        pm = _sc_scatter(pmsgs, colp2)
        hn, q, k, v, posp = _ln1_qkv(h, hm, pm, posp, ln1_wts)
        attn = _attention(q, k.reshape(NB, RB, DM), v.reshape(NB, RB, DM),
                          br, bc3, jb)
        h = _ff(hn, attn, ff_wts)

    return h[:N], posp[:N, :3]


# speedup vs baseline: 1.0982x; 1.0982x over previous
import functools

import jax
import jax.numpy as jnp
from jax import lax
from jax.experimental import pallas as pl
from jax.experimental.pallas import tpu as pltpu
from jax.experimental.pallas import tpu_sc as plsc

N = 10000
E = 160000
DM = 128
PD = 4
NHEADS = 8
HD = 16
NGRAPH = 64

NPAD = 10240
EPAD = 163840
NW = 32
CHK = 128
CPW = EPAD // (NW * CHK)
RB = 512
NB = NPAD // RB
EB = 512
STRIPE = NPAD // 16

_f32 = jnp.float32


def _sigmoid(x):
    return 1.0 / (1.0 + jnp.exp(-x))


def _gelu_tanh(x):
    return 0.5 * x * (1.0 + jnp.tanh(0.7978845608028654 * (x + 0.044715 * x * x * x)))


TW = 2 * DM


def _build_sc_kernels():
    mesh = plsc.VectorSubcoreMesh(core_axis_name="c", subcore_axis_name="s")

    GCHK = 64
    GPW = EPAD // (NW * GCHK)
    EPW = GPW * GCHK

    @functools.partial(
        pl.kernel,
        mesh=mesh,
        out_type=[
            jax.ShapeDtypeStruct((EPAD, TW), _f32),
            jax.ShapeDtypeStruct((EPAD, TW), _f32),
        ],
        scratch_types=[
            pltpu.VMEM((EPW,), jnp.int32),
            pltpu.VMEM((EPW,), jnp.int32),
            pltpu.VMEM((2, GCHK, TW), _f32),
            pltpu.VMEM((2, GCHK, TW), _f32),
            pltpu.SemaphoreType.DMA,
            pltpu.SemaphoreType.DMA,
            pltpu.SemaphoreType.DMA,
            pltpu.SemaphoreType.DMA,
        ],
    )
    def gather(tbl, rowi, coli, gr, gc, idr, idc, bufr, bufc, g0, g1, w0, w1):
        wid = lax.axis_index("s") * 2 + lax.axis_index("c")
        base = wid * EPW
        gsem = (g0, g1)
        wsem = (w0, w1)
        pltpu.sync_copy(rowi.at[pl.ds(base, EPW)], idr)
        pltpu.sync_copy(coli.at[pl.ds(base, EPW)], idc)

        def issue(j, slot):
            pltpu.async_copy(tbl.at[idr.at[pl.ds(j * GCHK, GCHK)]],
                             bufr.at[slot], gsem[slot])
            pltpu.async_copy(tbl.at[idc.at[pl.ds(j * GCHK, GCHK)]],
                             bufc.at[slot], gsem[slot])

        def drain(j, slot):
            pltpu.make_async_copy(tbl.at[idr.at[pl.ds(j * GCHK, GCHK)]],
                                  bufr.at[slot], gsem[slot]).wait()
            pltpu.make_async_copy(tbl.at[idc.at[pl.ds(j * GCHK, GCHK)]],
                                  bufc.at[slot], gsem[slot]).wait()

        def issue_wb(j, slot):
            off = base + j * GCHK
            pltpu.async_copy(bufr.at[slot], gr.at[pl.ds(off, GCHK)], wsem[slot])
            pltpu.async_copy(bufc.at[slot], gc.at[pl.ds(off, GCHK)], wsem[slot])

        def drain_wb(j, slot):
            off = base + j * GCHK
            pltpu.make_async_copy(bufr.at[slot], gr.at[pl.ds(off, GCHK)],
                                  wsem[slot]).wait()
            pltpu.make_async_copy(bufc.at[slot], gc.at[pl.ds(off, GCHK)],
                                  wsem[slot]).wait()

        issue(0, 0)

        def step(ci, slot):
            j = ci * 2 + slot

            @pl.when(j + 1 < GPW)
            def _():
                @pl.when(j >= 1)
                def _():
                    drain_wb(j - 1, 1 - slot)

                issue(j + 1, 1 - slot)

            drain(j, slot)
            issue_wb(j, slot)

        def body(ci, carry):
            step(ci, 0)
            step(ci, 1)
            return carry

        lax.fori_loop(0, GPW // 2, body, 0)
        drain_wb(GPW - 2, 0)
        drain_wb(GPW - 1, 1)

    HALF = NPAD // 2
    ACCR = 5248
    ZST = ACCR // 16
    OST = HALF // 16
    CPT = EPAD // (16 * CHK)

    @functools.partial(
        pl.kernel,
        mesh=mesh,
        out_type=[jax.ShapeDtypeStruct((NPAD, DM), _f32)],
        scratch_types=[
            pltpu.VMEM((CPT, CHK), jnp.int32),
            pltpu.VMEM((2, CHK, DM), _f32),
            pltpu.VMEM((ZST, DM), _f32),
            pltpu.VMEM_SHARED((ACCR, DM), _f32),
            pltpu.SemaphoreType.DMA,
            pltpu.SemaphoreType.DMA,
            pltpu.SemaphoreType.DMA,
            pltpu.SemaphoreType.DMA,
        ],
    )
    def scatter(msgs, coli, out, idx, sbuf, zbuf, acc, sem0, sem1, a0, a1):
        cid = lax.axis_index("c")
        sid = lax.axis_index("s")
        lo = cid * HALF
        sems = (sem0, sem1)
        asem = (a0, a1)
        pltpu.sync_copy(msgs.at[pl.ds(E, ZST)], zbuf)
        pltpu.sync_copy(zbuf, acc.at[pl.ds(sid * ZST, ZST)])
        pltpu.sync_copy(coli.at[pl.ds(sid * CPT, CPT)], idx)

        def remap(j, carry):
            r = idx.at[j]
            for g in range(CHK // 16):
                v = r[pl.ds(g * 16, 16)] - lo
                inr = (v >= 0) & (v < HALF)
                r[pl.ds(g * 16, 16)] = jnp.where(inr, v, HALF)
            return carry

        lax.fori_loop(0, CPT, remap, 0)
        plsc.subcore_barrier()

        def issue(j, slot):
            pltpu.async_copy(msgs.at[pl.ds((sid * CPT + j) * CHK, CHK)],
                             sbuf.at[slot], sems[slot])

        def drain(j, slot):
            pltpu.make_async_copy(msgs.at[pl.ds((sid * CPT + j) * CHK, CHK)],
                                  sbuf.at[slot], sems[slot]).wait()

        def issue_add(j, slot):
            pltpu.async_copy(sbuf.at[slot], acc.at[idx.at[j]], asem[slot],
                             add=True)

        def drain_add(j, slot):
            pltpu.make_async_copy(sbuf.at[slot], acc.at[idx.at[j]],
                                  asem[slot]).wait()

        issue(0, 0)

        def step(ci, slot):
            j = ci * 2 + slot

            @pl.when(j + 1 < CPT)
            def _():
                @pl.when(j >= 1)
                def _():
                    drain_add(j - 1, 1 - slot)

                issue(j + 1, 1 - slot)

            drain(j, slot)
            issue_add(j, slot)

        def body(ci, carry):
            step(ci, 0)
            step(ci, 1)
            return carry

        lax.fori_loop(0, CPT // 2, body, 0)
        drain_add(CPT - 2, 0)
        drain_add(CPT - 1, 1)
        plsc.subcore_barrier()
        pltpu.sync_copy(acc.at[pl.ds(sid * OST, OST)],
                        out.at[pl.ds(lo + sid * OST, OST)])

    return gather, scatter


_SC_CACHE = {}


def _sc_gather(tbl, rowp, colp):
    if "k" not in _SC_CACHE:
        _SC_CACHE["k"] = _build_sc_kernels()
    return _SC_CACHE["k"][0](tbl, rowp, colp)


def _sc_scatter(msgs, colp2):
    if "k" not in _SC_CACHE:
        _SC_CACHE["k"] = _build_sc_kernels()
    out = _SC_CACHE["k"][1](msgs, colp2)
    if isinstance(out, (tuple, list)):
        out = out[0]
    return out


def _edge_body(gr_ref, gc_ref, w1a, w1b, w1c, b1, w2, b2, pw1, pb1,
               pw2, pb2, m_ref, pm_ref):
    gr = gr_ref[...]
    gc = gc_ref[...]
    hr = gr[:, :DM]
    hc = gc[:, :DM]
    rel = gr[:, DM:DM + PD] - gc[:, DM:DM + PD]
    dist = jnp.sqrt(jnp.sum(rel * rel, axis=1, keepdims=True))
    z = (jnp.dot(hr, w1a[...], preferred_element_type=_f32)
         + jnp.dot(hc, w1b[...], preferred_element_type=_f32)
         + dist * w1c[...] + b1[...])
    t = z * _sigmoid(z)
    emsg = jnp.dot(t, w2[...], preferred_element_type=_f32) + b2[...]
    u = emsg @ pw1[...] + pb1[...]
    u = u * _sigmoid(u)
    coeff = jnp.dot(u, pw2[...], preferred_element_type=_f32) + pb2[...]
    gid = pl.program_id(0) * EB + lax.broadcasted_iota(jnp.int32, (EB, 1), 0)
    valid = gid < E
    m_ref[...] = jnp.where(valid, hr * emsg, 0.0)
    pm_ref[...] = jnp.where(valid, rel * coeff, 0.0)


def _edge_mlp(gr, gc, wts):
    full = lambda shape: pl.BlockSpec(shape, lambda i: (0,) * len(shape))
    return pl.pallas_call(
        _edge_body,
        grid=(EPAD // EB,),
        in_specs=[
            pl.BlockSpec((EB, TW), lambda i: (i, 0)),
            pl.BlockSpec((EB, TW), lambda i: (i, 0)),
            full((DM, DM)), full((DM, DM)), full((1, DM)), full((1, DM)),
            full((DM, DM)), full((1, DM)),
            full((DM, DM)), full((1, DM)), full((DM, 1)), full((1, 1)),
        ],
        out_specs=[pl.BlockSpec((EB, DM), lambda i: (i, 0)),
                   pl.BlockSpec((EB, PD), lambda i: (i, 0))],
        out_shape=[jax.ShapeDtypeStruct((EPAD, DM), _f32),
                   jax.ShapeDtypeStruct((EPAD, PD), _f32)],
    )(gr, gc, *wts)


EC = EPAD // EB


def _ln1_body(eb_ref, h_ref, sp_ref, pm4_ref, col3_ref, posp_ref,
              wq, wk, wv, bq, bk, bv, g1, b1g,
              hn_ref, q_ref, k_ref, v_ref, pos_ref):
    i = pl.program_id(0)
    t = h_ref[...] + sp_ref[...]
    mu = jnp.mean(t, axis=1, keepdims=True)
    d = t - mu
    var = jnp.mean(d * d, axis=1, keepdims=True)
    hn = d * lax.rsqrt(var + 1e-5) * g1[...] + b1g[...]
    hn_ref[...] = hn
    q_ref[...] = jnp.dot(hn, wq[...], preferred_element_type=_f32) + bq[...]
    k_ref[...] = jnp.dot(hn, wk[...], preferred_element_type=_f32) + bk[...]
    v_ref[...] = jnp.dot(hn, wv[...], preferred_element_type=_f32) + bv[...]
    nodeids = i * RB + lax.broadcasted_iota(jnp.int32, (RB, 1), 0)

    def body(j, accT):
        mask = jnp.where(nodeids == col3_ref[j], 1.0, 0.0)
        return accT + lax.dot_general(pm4_ref[j], mask, (((1,), (1,)), ((), ())),
                                      preferred_element_type=_f32)

    pmsumT = lax.fori_loop(eb_ref[i, 0], eb_ref[i, 1], body,
                           jnp.zeros((PD, RB), _f32))
    pos_ref[...] = posp_ref[...] + pmsumT


def _ln1_qkv(h, sp, pm4t, col3, pospT, eb, wts):
    full = lambda shape: pl.BlockSpec(shape, lambda i, eb: (0,) * len(shape))
    blk = pl.BlockSpec((RB, DM), lambda i, eb: (i, 0))
    pblk = pl.BlockSpec((PD, RB), lambda i, eb: (0, i))
    spec = pltpu.PrefetchScalarGridSpec(
        num_scalar_prefetch=1,
        grid=(NB,),
        in_specs=[
            blk,
            blk,
            full((EC, PD, EB)),
            full((EC, 1, EB)),
            pblk,
            full((DM, DM)), full((DM, DM)), full((DM, DM)),
            full((1, DM)), full((1, DM)), full((1, DM)),
            full((1, DM)), full((1, DM)),
        ],
        out_specs=[blk, blk, blk, blk, pblk],
    )
    return pl.pallas_call(
        _ln1_body,
        grid_spec=spec,
        out_shape=[
            jax.ShapeDtypeStruct((NPAD, DM), _f32),
            jax.ShapeDtypeStruct((NPAD, DM), _f32),
            jax.ShapeDtypeStruct((NPAD, DM), _f32),
            jax.ShapeDtypeStruct((NPAD, DM), _f32),
            jax.ShapeDtypeStruct((PD, NPAD), _f32),
        ],
    )(eb, h, sp, pm4t, col3, pospT, *wts)


def _attn_body(jb_ref, q_ref, k3_ref, v3_ref, br_ref, bc3_ref, o_ref):
    i = pl.program_id(0)
    jlo = jb_ref[i, 0]
    jhi = jb_ref[i, 1]
    q = q_ref[...]
    br = br_ref[...]

    carry = []
    for _ in range(NHEADS):
        carry.append(jnp.full((RB, 1), -1e30, _f32))
        carry.append(jnp.zeros((RB, 1), _f32))
        carry.append(jnp.zeros((RB, HD), _f32))
    carry = tuple(carry)

    def body(j, carry):
        kc = k3_ref[j]
        vc = v3_ref[j]
        bc = bc3_ref[j]
        mask = br == bc
        out = []
        for h in range(NHEADS):
            m_h = carry[3 * h]
            l_h = carry[3 * h + 1]
            a_h = carry[3 * h + 2]
            qh = q[:, h * HD:(h + 1) * HD]
            kh = kc[:, h * HD:(h + 1) * HD]
            vh = vc[:, h * HD:(h + 1) * HD]
            s = lax.dot_general(qh, kh, (((1,), (1,)), ((), ())),
                                preferred_element_type=_f32) * 0.25
            s = jnp.where(mask, s, -1e9)
            mnew = jnp.maximum(m_h, jnp.max(s, axis=1, keepdims=True))
            alpha = jnp.exp(m_h - mnew)
            p = jnp.exp(s - mnew)
            l_new = l_h * alpha + jnp.sum(p, axis=1, keepdims=True)
            a_new = a_h * alpha + jnp.dot(p, vh, preferred_element_type=_f32)
            out.extend([mnew, l_new, a_new])
        return tuple(out)

    carry = lax.fori_loop(jlo, jhi, body, carry)
    outs = [carry[3 * h + 2] / carry[3 * h + 1] for h in range(NHEADS)]
    o_ref[...] = jnp.concatenate(outs, axis=1)


def _attention(q, k3, v3, br, bc3, jb):
    spec = pltpu.PrefetchScalarGridSpec(
        num_scalar_prefetch=1,
        grid=(NB,),
        in_specs=[
            pl.BlockSpec((RB, DM), lambda i, jb: (i, 0)),
            pl.BlockSpec((NB, RB, DM), lambda i, jb: (0, 0, 0)),
            pl.BlockSpec((NB, RB, DM), lambda i, jb: (0, 0, 0)),
            pl.BlockSpec((RB, 1), lambda i, jb: (i, 0)),
            pl.BlockSpec((NB, 1, RB), lambda i, jb: (0, 0, 0)),
        ],
        out_specs=pl.BlockSpec((RB, DM), lambda i, jb: (i, 0)),
    )
    return pl.pallas_call(
        _attn_body,
        grid_spec=spec,
        out_shape=jax.ShapeDtypeStruct((NPAD, DM), _f32),
    )(jb, q, k3, v3, br, bc3)


def _ff_body(hn_ref, a_ref, wo, bo, g2, b2g, w1, bb1, w2, bb2, out_ref):
    r = hn_ref[...] + jnp.dot(a_ref[...], wo[...], preferred_element_type=_f32) + bo[...]
    mu = jnp.mean(r, axis=1, keepdims=True)
    d = r - mu
    var = jnp.mean(d * d, axis=1, keepdims=True)
    h2 = d * lax.rsqrt(var + 1e-5) * g2[...] + b2g[...]
    f = _gelu_tanh(jnp.dot(h2, w1[...], preferred_element_type=_f32) + bb1[...])
    out_ref[...] = h2 + jnp.dot(f, w2[...], preferred_element_type=_f32) + bb2[...]


def _ff(hn, attn, wts):
    full = lambda shape: pl.BlockSpec(shape, lambda i: (0,) * len(shape))
    blk = pl.BlockSpec((RB, DM), lambda i: (i, 0))
    return pl.pallas_call(
        _ff_body,
        grid=(NB,),
        in_specs=[
            blk, blk,
            full((DM, DM)), full((1, DM)), full((1, DM)), full((1, DM)),
            full((DM, 4 * DM)), full((1, 4 * DM)),
            full((4 * DM, DM)), full((1, DM)),
        ],
        out_specs=blk,
        out_shape=jax.ShapeDtypeStruct((NPAD, DM), _f32),
    )(hn, attn, *wts)


def kernel(x, pos, edge_index, edge_attr, batch, params):
    del edge_attr
    row = edge_index[0].astype(jnp.int32)
    col = edge_index[1].astype(jnp.int32)
    perm = jnp.argsort(col)
    row = row[perm]
    col = col[perm]
    epad = EPAD - E
    rowp = jnp.concatenate([row, jnp.zeros((epad,), jnp.int32)])
    colp = jnp.concatenate([col, jnp.full((epad,), NPAD - 1, jnp.int32)])
    colp2 = colp.reshape(EPAD // CHK, CHK)
    col3 = colp.reshape(EC, 1, EB)
    estart = jnp.searchsorted(colp, jnp.arange(NB + 1, dtype=jnp.int32) * RB)
    eb = jnp.stack([estart[:NB] // EB,
                    (estart[1:] + EB - 1) // EB], axis=1).astype(jnp.int32)

    npd = NPAD - N
    h = jnp.concatenate([x, jnp.zeros((npd, DM), _f32)], axis=0)
    posp = jnp.concatenate(
        [jnp.pad(pos, ((0, 0), (0, PD - 3))), jnp.zeros((npd, PD), _f32)], axis=0)
    batch_pad = jnp.concatenate([batch.astype(jnp.int32),
                                 jnp.full((npd,), 127, jnp.int32)])

    starts = jnp.searchsorted(batch, jnp.arange(NGRAPH + 1)).astype(jnp.int32)
    cs = jnp.concatenate([starts[jnp.clip(batch, 0, NGRAPH - 1)],
                          jnp.full((npd,), N, jnp.int32)])
    ce = jnp.concatenate([starts[jnp.clip(batch, 0, NGRAPH - 1) + 1],
                          jnp.full((npd,), NPAD, jnp.int32)])
    r0 = jnp.arange(NB, dtype=jnp.int32) * RB
    jlo = cs[r0] // RB
    jhi = (ce[r0 + RB - 1] + RB - 1) // RB
    jb = jnp.stack([jlo, jhi], axis=1).astype(jnp.int32)

    br = batch_pad[:, None]
    bc3 = batch_pad.reshape(NB, 1, RB)
    ztail = jnp.zeros((NPAD, TW - DM - PD), _f32)

    for p in params:
        w1 = p['pe_W1']
        edge_wts = (
            w1[:, :DM].T, w1[:, DM:2 * DM].T, w1[:, 2 * DM:].T,
            p['pe_b1'][None], p['pe_W2'].T, p['pe_b2'][None],
            p['px_W1'].T, p['px_b1'][None], p['px_W2'].T, p['px_b2'][None],
        )
        wqkv = p['Wqkv']
        ln1_wts = (
            wqkv[:DM].T, wqkv[DM:2 * DM].T, wqkv[2 * DM:].T,
            p['bqkv'][None, :DM], p['bqkv'][None, DM:2 * DM], p['bqkv'][None, 2 * DM:],
            p['ln1_g'][None], p['ln1_b'][None],
        )
        ff_wts = (
            p['Wo'].T, p['bo'][None], p['ln2_g'][None], p['ln2_b'][None],
            p['ff_W1'].T, p['ff_b1'][None], p['ff_W2'].T, p['ff_b2'][None],
        )

        tbl = jnp.concatenate([h, posp, ztail], axis=1)
        gr, gc = _sc_gather(tbl, rowp, colp)
        msgs, pm4 = _edge_mlp(gr, gc, edge_wts)
        hm = _sc_scatter(msgs, colp2)
        pm4t = pm4.reshape(EC, EB, PD).transpose(0, 2, 1)
        hn, q, k, v, posT = _ln1_qkv(h, hm, pm4t, col3, posp.T, eb, ln1_wts)
        posp = posT.T
        attn = _attention(q, k.reshape(NB, RB, DM), v.reshape(NB, RB, DM),
                          br, bc3, jb)
        h = _ff(hn, attn, ff_wts)

    return h[:N], posp[:N, :3]


# speedup vs baseline: 1.1096x; 1.0104x over previous
import functools

import jax
import jax.numpy as jnp
from jax import lax
from jax.experimental import pallas as pl
from jax.experimental.pallas import tpu as pltpu
from jax.experimental.pallas import tpu_sc as plsc

N = 10000
E = 160000
DM = 128
PD = 4
NHEADS = 8
HD = 16
NGRAPH = 64

NPAD = 10240
EPAD = 163840
NW = 32
CHK = 128
CPW = EPAD // (NW * CHK)
RB = 512
NB = NPAD // RB
EB = 512
STRIPE = NPAD // 16

_f32 = jnp.float32


def _sigmoid(x):
    return 1.0 / (1.0 + jnp.exp(-x))


def _gelu_tanh(x):
    return 0.5 * x * (1.0 + jnp.tanh(0.7978845608028654 * (x + 0.044715 * x * x * x)))


TW = 2 * DM


def _build_sc_kernels():
    mesh = plsc.VectorSubcoreMesh(core_axis_name="c", subcore_axis_name="s")

    GCHK = 64
    GPW = EPAD // (NW * GCHK)
    EPW = GPW * GCHK

    @functools.partial(
        pl.kernel,
        mesh=mesh,
        out_type=[
            jax.ShapeDtypeStruct((EPAD, TW), _f32),
        ],
        scratch_types=[
            pltpu.VMEM((EPW,), jnp.int32),
            pltpu.VMEM((2, GCHK, TW), _f32),
            pltpu.SemaphoreType.DMA,
            pltpu.SemaphoreType.DMA,
            pltpu.SemaphoreType.DMA,
            pltpu.SemaphoreType.DMA,
        ],
    )
    def gather(tbl, rowi, gr, idr, bufr, g0, g1, w0, w1):
        wid = lax.axis_index("s") * 2 + lax.axis_index("c")
        base = wid * EPW
        gsem = (g0, g1)
        wsem = (w0, w1)
        pltpu.sync_copy(rowi.at[pl.ds(base, EPW)], idr)

        def issue(j, slot):
            pltpu.async_copy(tbl.at[idr.at[pl.ds(j * GCHK, GCHK)]],
                             bufr.at[slot], gsem[slot])

        def drain(j, slot):
            pltpu.make_async_copy(tbl.at[idr.at[pl.ds(j * GCHK, GCHK)]],
                                  bufr.at[slot], gsem[slot]).wait()

        def issue_wb(j, slot):
            off = base + j * GCHK
            pltpu.async_copy(bufr.at[slot], gr.at[pl.ds(off, GCHK)], wsem[slot])

        def drain_wb(j, slot):
            off = base + j * GCHK
            pltpu.make_async_copy(bufr.at[slot], gr.at[pl.ds(off, GCHK)],
                                  wsem[slot]).wait()

        issue(0, 0)

        def step(ci, slot):
            j = ci * 2 + slot

            @pl.when(j + 1 < GPW)
            def _():
                @pl.when(j >= 1)
                def _():
                    drain_wb(j - 1, 1 - slot)

                issue(j + 1, 1 - slot)

            drain(j, slot)
            issue_wb(j, slot)

        def body(ci, carry):
            step(ci, 0)
            step(ci, 1)
            return carry

        lax.fori_loop(0, GPW // 2, body, 0)
        drain_wb(GPW - 2, 0)
        drain_wb(GPW - 1, 1)

    HALF = NPAD // 2
    ACCR = 5248
    ZST = ACCR // 16
    OST = HALF // 16
    CPT = EPAD // (16 * CHK)

    @functools.partial(
        pl.kernel,
        mesh=mesh,
        out_type=[jax.ShapeDtypeStruct((NPAD, DM), _f32)],
        scratch_types=[
            pltpu.VMEM((CPT, CHK), jnp.int32),
            pltpu.VMEM((2, CHK, DM), _f32),
            pltpu.VMEM((ZST, DM), _f32),
            pltpu.VMEM_SHARED((ACCR, DM), _f32),
            pltpu.SemaphoreType.DMA,
            pltpu.SemaphoreType.DMA,
            pltpu.SemaphoreType.DMA,
            pltpu.SemaphoreType.DMA,
        ],
    )
    def scatter(msgs, coli, out, idx, sbuf, zbuf, acc, sem0, sem1, a0, a1):
        cid = lax.axis_index("c")
        sid = lax.axis_index("s")
        lo = cid * HALF
        sems = (sem0, sem1)
        asem = (a0, a1)
        pltpu.sync_copy(msgs.at[pl.ds(E, ZST)], zbuf)
        pltpu.sync_copy(zbuf, acc.at[pl.ds(sid * ZST, ZST)])
        pltpu.sync_copy(coli.at[pl.ds(sid * CPT, CPT)], idx)

        def remap(j, carry):
            r = idx.at[j]
            for g in range(CHK // 16):
                v = r[pl.ds(g * 16, 16)] - lo
                inr = (v >= 0) & (v < HALF)
                r[pl.ds(g * 16, 16)] = jnp.where(inr, v, HALF)
            return carry

        lax.fori_loop(0, CPT, remap, 0)
        plsc.subcore_barrier()

        def issue(j, slot):
            pltpu.async_copy(msgs.at[pl.ds((sid * CPT + j) * CHK, CHK)],
                             sbuf.at[slot], sems[slot])

        def drain(j, slot):
            pltpu.make_async_copy(msgs.at[pl.ds((sid * CPT + j) * CHK, CHK)],
                                  sbuf.at[slot], sems[slot]).wait()

        def issue_add(j, slot):
            pltpu.async_copy(sbuf.at[slot], acc.at[idx.at[j]], asem[slot],
                             add=True)

        def drain_add(j, slot):
            pltpu.make_async_copy(sbuf.at[slot], acc.at[idx.at[j]],
                                  asem[slot]).wait()

        issue(0, 0)

        def step(ci, slot):
            j = ci * 2 + slot

            @pl.when(j + 1 < CPT)
            def _():
                @pl.when(j >= 1)
                def _():
                    drain_add(j - 1, 1 - slot)

                issue(j + 1, 1 - slot)

            drain(j, slot)
            issue_add(j, slot)

        def body(ci, carry):
            step(ci, 0)
            step(ci, 1)
            return carry

        lax.fori_loop(0, CPT // 2, body, 0)
        drain_add(CPT - 2, 0)
        drain_add(CPT - 1, 1)
        plsc.subcore_barrier()
        pltpu.sync_copy(acc.at[pl.ds(sid * OST, OST)],
                        out.at[pl.ds(lo + sid * OST, OST)])

    return gather, scatter


_SC_CACHE = {}


def _sc_gather(tbl, rowp):
    if "k" not in _SC_CACHE:
        _SC_CACHE["k"] = _build_sc_kernels()
    out = _SC_CACHE["k"][0](tbl, rowp)
    if isinstance(out, (tuple, list)):
        out = out[0]
    return out


def _sc_scatter(msgs, colp2):
    if "k" not in _SC_CACHE:
        _SC_CACHE["k"] = _build_sc_kernels()
    out = _SC_CACHE["k"][1](msgs, colp2)
    if isinstance(out, (tuple, list)):
        out = out[0]
    return out


def _edge_body(gr_ref, tbl_ref, colc_ref, w1a, w1b, w1c, b1, w2, b2, pw1, pb1,
               pw2, pb2, m_ref, pm_ref):
    gr = gr_ref[...]
    colc = colc_ref[...]
    wlo = jnp.min(colc) // RB
    whi = jnp.max(colc) // RB

    def wbody(w, acc):
        woff = pl.multiple_of(w * RB, RB)
        win = tbl_ref[pl.ds(woff, RB), :]
        winids = w * RB + lax.broadcasted_iota(jnp.int32, (1, RB), 1)
        maskf = jnp.where(colc == winids, 1.0, 0.0)
        return acc + jnp.dot(maskf, win, preferred_element_type=_f32)

    gc = lax.fori_loop(wlo, whi + 1, wbody, jnp.zeros((EB, TW), _f32))
    hr = gr[:, :DM]
    hc = gc[:, :DM]
    rel = gr[:, DM:DM + PD] - gc[:, DM:DM + PD]
    dist = jnp.sqrt(jnp.sum(rel * rel, axis=1, keepdims=True))
    z = (jnp.dot(hr, w1a[...], preferred_element_type=_f32)
         + jnp.dot(hc, w1b[...], preferred_element_type=_f32)
         + dist * w1c[...] + b1[...])
    t = z * _sigmoid(z)
    emsg = jnp.dot(t, w2[...], preferred_element_type=_f32) + b2[...]
    u = emsg @ pw1[...] + pb1[...]
    u = u * _sigmoid(u)
    coeff = jnp.dot(u, pw2[...], preferred_element_type=_f32) + pb2[...]
    gid = pl.program_id(0) * EB + lax.broadcasted_iota(jnp.int32, (EB, 1), 0)
    valid = gid < E
    m_ref[...] = jnp.where(valid, hr * emsg, 0.0)
    pm_ref[...] = jnp.where(valid, rel * coeff, 0.0)


def _edge_mlp(gr, tbl, colv, wts):
    full = lambda shape: pl.BlockSpec(shape, lambda i: (0,) * len(shape))
    return pl.pallas_call(
        _edge_body,
        grid=(EPAD // EB,),
        in_specs=[
            pl.BlockSpec((EB, TW), lambda i: (i, 0)),
            full((NPAD, TW)),
            pl.BlockSpec((EB, 1), lambda i: (i, 0)),
            full((DM, DM)), full((DM, DM)), full((1, DM)), full((1, DM)),
            full((DM, DM)), full((1, DM)),
            full((DM, DM)), full((1, DM)), full((DM, 1)), full((1, 1)),
        ],
        out_specs=[pl.BlockSpec((EB, DM), lambda i: (i, 0)),
                   pl.BlockSpec((EB, PD), lambda i: (i, 0))],
        out_shape=[jax.ShapeDtypeStruct((EPAD, DM), _f32),
                   jax.ShapeDtypeStruct((EPAD, PD), _f32)],
    )(gr, tbl, colv, *wts)


EC = EPAD // EB


def _ln1_body(eb_ref, h_ref, sp_ref, pm4_ref, col3_ref, posp_ref,
              wq, wk, wv, bq, bk, bv, g1, b1g,
              hn_ref, q_ref, k_ref, v_ref, pos_ref):
    i = pl.program_id(0)
    t = h_ref[...] + sp_ref[...]
    mu = jnp.mean(t, axis=1, keepdims=True)
    d = t - mu
    var = jnp.mean(d * d, axis=1, keepdims=True)
    hn = d * lax.rsqrt(var + 1e-5) * g1[...] + b1g[...]
    hn_ref[...] = hn
    q_ref[...] = jnp.dot(hn, wq[...], preferred_element_type=_f32) + bq[...]
    k_ref[...] = jnp.dot(hn, wk[...], preferred_element_type=_f32) + bk[...]
    v_ref[...] = jnp.dot(hn, wv[...], preferred_element_type=_f32) + bv[...]
    nodeids = i * RB + lax.broadcasted_iota(jnp.int32, (RB, 1), 0)

    def body(j, accT):
        mask = jnp.where(nodeids == col3_ref[j], 1.0, 0.0)
        return accT + lax.dot_general(pm4_ref[j], mask, (((1,), (1,)), ((), ())),
                                      preferred_element_type=_f32)

    pmsumT = lax.fori_loop(eb_ref[i, 0], eb_ref[i, 1], body,
                           jnp.zeros((PD, RB), _f32))
    pos_ref[...] = posp_ref[...] + pmsumT


def _ln1_qkv(h, sp, pm4t, col3, pospT, eb, wts):
    full = lambda shape: pl.BlockSpec(shape, lambda i, eb: (0,) * len(shape))
    blk = pl.BlockSpec((RB, DM), lambda i, eb: (i, 0))
    pblk = pl.BlockSpec((PD, RB), lambda i, eb: (0, i))
    spec = pltpu.PrefetchScalarGridSpec(
        num_scalar_prefetch=1,
        grid=(NB,),
        in_specs=[
            blk,
            blk,
            full((EC, PD, EB)),
            full((EC, 1, EB)),
            pblk,
            full((DM, DM)), full((DM, DM)), full((DM, DM)),
            full((1, DM)), full((1, DM)), full((1, DM)),
            full((1, DM)), full((1, DM)),
        ],
        out_specs=[blk, blk, blk, blk, pblk],
    )
    return pl.pallas_call(
        _ln1_body,
        grid_spec=spec,
        out_shape=[
            jax.ShapeDtypeStruct((NPAD, DM), _f32),
            jax.ShapeDtypeStruct((NPAD, DM), _f32),
            jax.ShapeDtypeStruct((NPAD, DM), _f32),
            jax.ShapeDtypeStruct((NPAD, DM), _f32),
            jax.ShapeDtypeStruct((PD, NPAD), _f32),
        ],
    )(eb, h, sp, pm4t, col3, pospT, *wts)


def _attn_body(jb_ref, q_ref, k3_ref, v3_ref, br_ref, bc3_ref, o_ref):
    i = pl.program_id(0)
    jlo = jb_ref[i, 0]
    jhi = jb_ref[i, 1]
    q = q_ref[...]
    br = br_ref[...]

    carry = []
    for _ in range(NHEADS):
        carry.append(jnp.full((RB, 1), -1e30, _f32))
        carry.append(jnp.zeros((RB, 1), _f32))
        carry.append(jnp.zeros((RB, HD), _f32))
    carry = tuple(carry)

    def body(j, carry):
        kc = k3_ref[j]
        vc = v3_ref[j]
        bc = bc3_ref[j]
        mask = br == bc
        out = []
        for h in range(NHEADS):
            m_h = carry[3 * h]
            l_h = carry[3 * h + 1]
            a_h = carry[3 * h + 2]
            qh = q[:, h * HD:(h + 1) * HD]
            kh = kc[:, h * HD:(h + 1) * HD]
            vh = vc[:, h * HD:(h + 1) * HD]
            s = lax.dot_general(qh, kh, (((1,), (1,)), ((), ())),
                                preferred_element_type=_f32) * 0.25
            s = jnp.where(mask, s, -1e9)
            mnew = jnp.maximum(m_h, jnp.max(s, axis=1, keepdims=True))
            alpha = jnp.exp(m_h - mnew)
            p = jnp.exp(s - mnew)
            l_new = l_h * alpha + jnp.sum(p, axis=1, keepdims=True)
            a_new = a_h * alpha + jnp.dot(p, vh, preferred_element_type=_f32)
            out.extend([mnew, l_new, a_new])
        return tuple(out)

    carry = lax.fori_loop(jlo, jhi, body, carry)
    outs = [carry[3 * h + 2] / carry[3 * h + 1] for h in range(NHEADS)]
    o_ref[...] = jnp.concatenate(outs, axis=1)


def _attention(q, k3, v3, br, bc3, jb):
    spec = pltpu.PrefetchScalarGridSpec(
        num_scalar_prefetch=1,
        grid=(NB,),
        in_specs=[
            pl.BlockSpec((RB, DM), lambda i, jb: (i, 0)),
            pl.BlockSpec((NB, RB, DM), lambda i, jb: (0, 0, 0)),
            pl.BlockSpec((NB, RB, DM), lambda i, jb: (0, 0, 0)),
            pl.BlockSpec((RB, 1), lambda i, jb: (i, 0)),
            pl.BlockSpec((NB, 1, RB), lambda i, jb: (0, 0, 0)),
        ],
        out_specs=pl.BlockSpec((RB, DM), lambda i, jb: (i, 0)),
    )
    return pl.pallas_call(
        _attn_body,
        grid_spec=spec,
        out_shape=jax.ShapeDtypeStruct((NPAD, DM), _f32),
    )(jb, q, k3, v3, br, bc3)


def _ff_body(hn_ref, a_ref, wo, bo, g2, b2g, w1, bb1, w2, bb2, out_ref):
    r = hn_ref[...] + jnp.dot(a_ref[...], wo[...], preferred_element_type=_f32) + bo[...]
    mu = jnp.mean(r, axis=1, keepdims=True)
    d = r - mu
    var = jnp.mean(d * d, axis=1, keepdims=True)
    h2 = d * lax.rsqrt(var + 1e-5) * g2[...] + b2g[...]
    f = _gelu_tanh(jnp.dot(h2, w1[...], preferred_element_type=_f32) + bb1[...])
    out_ref[...] = h2 + jnp.dot(f, w2[...], preferred_element_type=_f32) + bb2[...]


def _ff(hn, attn, wts):
    full = lambda shape: pl.BlockSpec(shape, lambda i: (0,) * len(shape))
    blk = pl.BlockSpec((RB, DM), lambda i: (i, 0))
    return pl.pallas_call(
        _ff_body,
        grid=(NB,),
        in_specs=[
            blk, blk,
            full((DM, DM)), full((1, DM)), full((1, DM)), full((1, DM)),
            full((DM, 4 * DM)), full((1, 4 * DM)),
            full((4 * DM, DM)), full((1, DM)),
        ],
        out_specs=blk,
        out_shape=jax.ShapeDtypeStruct((NPAD, DM), _f32),
    )(hn, attn, *wts)


def kernel(x, pos, edge_index, edge_attr, batch, params):
    del edge_attr
    row = edge_index[0].astype(jnp.int32)
    col = edge_index[1].astype(jnp.int32)
    perm = jnp.argsort(col)
    row = row[perm]
    col = col[perm]
    epad = EPAD - E
    rowp = jnp.concatenate([row, jnp.zeros((epad,), jnp.int32)])
    colp = jnp.concatenate([col, jnp.full((epad,), NPAD - 1, jnp.int32)])
    colp2 = colp.reshape(EPAD // CHK, CHK)
    col3 = colp.reshape(EC, 1, EB)
    colv = colp[:, None]
    estart = jnp.searchsorted(colp, jnp.arange(NB + 1, dtype=jnp.int32) * RB)
    eb = jnp.stack([estart[:NB] // EB,
                    (estart[1:] + EB - 1) // EB], axis=1).astype(jnp.int32)

    npd = NPAD - N
    h = jnp.concatenate([x, jnp.zeros((npd, DM), _f32)], axis=0)
    posp = jnp.concatenate(
        [jnp.pad(pos, ((0, 0), (0, PD - 3))), jnp.zeros((npd, PD), _f32)], axis=0)
    batch_pad = jnp.concatenate([batch.astype(jnp.int32),
                                 jnp.full((npd,), 127, jnp.int32)])

    starts = jnp.searchsorted(batch, jnp.arange(NGRAPH + 1)).astype(jnp.int32)
    cs = jnp.concatenate([starts[jnp.clip(batch, 0, NGRAPH - 1)],
                          jnp.full((npd,), N, jnp.int32)])
    ce = jnp.concatenate([starts[jnp.clip(batch, 0, NGRAPH - 1) + 1],
                          jnp.full((npd,), NPAD, jnp.int32)])
    r0 = jnp.arange(NB, dtype=jnp.int32) * RB
    jlo = cs[r0] // RB
    jhi = (ce[r0 + RB - 1] + RB - 1) // RB
    jb = jnp.stack([jlo, jhi], axis=1).astype(jnp.int32)

    br = batch_pad[:, None]
    bc3 = batch_pad.reshape(NB, 1, RB)
    ztail = jnp.zeros((NPAD, TW - DM - PD), _f32)

    for p in params:
        w1 = p['pe_W1']
        edge_wts = (
            w1[:, :DM].T, w1[:, DM:2 * DM].T, w1[:, 2 * DM:].T,
            p['pe_b1'][None], p['pe_W2'].T, p['pe_b2'][None],
            p['px_W1'].T, p['px_b1'][None], p['px_W2'].T, p['px_b2'][None],
        )
        wqkv = p['Wqkv']
        ln1_wts = (
            wqkv[:DM].T, wqkv[DM:2 * DM].T, wqkv[2 * DM:].T,
            p['bqkv'][None, :DM], p['bqkv'][None, DM:2 * DM], p['bqkv'][None, 2 * DM:],
            p['ln1_g'][None], p['ln1_b'][None],
        )
        ff_wts = (
            p['Wo'].T, p['bo'][None], p['ln2_g'][None], p['ln2_b'][None],
            p['ff_W1'].T, p['ff_b1'][None], p['ff_W2'].T, p['ff_b2'][None],
        )

        tbl = jnp.concatenate([h, posp, ztail], axis=1)
        gr = _sc_gather(tbl, rowp)
        msgs, pm4 = _edge_mlp(gr, tbl, colv, edge_wts)
        hm = _sc_scatter(msgs, colp2)
        pm4t = pm4.reshape(EC, EB, PD).transpose(0, 2, 1)
        hn, q, k, v, posT = _ln1_qkv(h, hm, pm4t, col3, posp.T, eb, ln1_wts)
        posp = posT.T
        attn = _attention(q, k.reshape(NB, RB, DM), v.reshape(NB, RB, DM),
                          br, bc3, jb)
        h = _ff(hn, attn, ff_wts)

    return h[:N], posp[:N, :3]
